# initial kernel scaffold (unmeasured)
import jax
import jax.numpy as jnp
from jax import lax
from jax.experimental import pallas as pl
from jax.experimental.pallas import tpu as pltpu

N_DEV = 8


def kernel(x, W1, W2):
    m, k_loc = x.shape
    _, d_model = W1.shape
    _, f_loc = W2.shape
    mc = m // N_DEV

    def body(x_ref, w1_ref, w2_ref, out_ref,
             w1b, w2b, rs_buf, ag_buf,
             rs_send_sems, rs_recv_sems, ag_send_sems, ag_recv_sems):
        d = lax.axis_index("i")
        right = lax.rem(d + 1, N_DEV)

        w1b[...] = w1_ref[...].astype(jnp.bfloat16)
        w2b[...] = w2_ref[...].astype(jnp.bfloat16)

        def partial_chunk(c):
            xs = x_ref[pl.ds(c * mc, mc), :].astype(jnp.bfloat16)
            return lax.dot(xs, w1b[...], preferred_element_type=jnp.float32)

        rs_buf[N_DEV - 1] = partial_chunk(
            lax.rem(d + N_DEV - 1, N_DEV)).astype(jnp.bfloat16)

        h_mine = None
        for s in range(N_DEV - 1):
            send_slot = N_DEV - 1 if s == 0 else s - 1
            rdma = pltpu.make_async_remote_copy(
                src_ref=rs_buf.at[send_slot],
                dst_ref=rs_buf.at[s],
                send_sem=rs_send_sems.at[s],
                recv_sem=rs_recv_sems.at[s],
                device_id=(right,),
                device_id_type=pl.DeviceIdType.MESH,
            )
            rdma.start()
            rdma.wait()
            c = lax.rem(d + 2 * N_DEV - 2 - s, N_DEV)
            p = partial_chunk(c)
            if s < N_DEV - 2:
                rs_buf[s] = (p + rs_buf[s].astype(jnp.float32)).astype(
                    jnp.bfloat16)
            else:
                h_mine = p + rs_buf[s].astype(jnp.float32)

        ag_buf[pl.ds(d * mc, mc), :] = h_mine.astype(jnp.bfloat16)

        out_ref[pl.ds(d * mc, mc), :] = lax.dot(
            h_mine.astype(jnp.bfloat16), w2b[...],
            preferred_element_type=jnp.float32)

        for t in range(N_DEV - 1):
            cs = lax.rem(d + N_DEV - t, N_DEV)
            cr = lax.rem(d + 2 * N_DEV - 1 - t, N_DEV)
            rdma = pltpu.make_async_remote_copy(
                src_ref=ag_buf.at[pl.ds(cs * mc, mc)],
                dst_ref=ag_buf.at[pl.ds(cs * mc, mc)],
                send_sem=ag_send_sems.at[t],
                recv_sem=ag_recv_sems.at[t],
                device_id=(right,),
                device_id_type=pl.DeviceIdType.MESH,
            )
            rdma.start()
            rdma.wait()
            out_ref[pl.ds(cr * mc, mc), :] = lax.dot(
                ag_buf[pl.ds(cr * mc, mc), :], w2b[...],
                preferred_element_type=jnp.float32)

    return pl.pallas_call(
        body,
        out_shape=jax.ShapeDtypeStruct((m, f_loc), jnp.float32),
        in_specs=[
            pl.BlockSpec(memory_space=pltpu.VMEM),
            pl.BlockSpec(memory_space=pltpu.VMEM),
            pl.BlockSpec(memory_space=pltpu.VMEM),
        ],
        out_specs=pl.BlockSpec(memory_space=pltpu.VMEM),
        scratch_shapes=[
            pltpu.VMEM((k_loc, d_model), jnp.bfloat16),
            pltpu.VMEM((d_model, f_loc), jnp.bfloat16),
            pltpu.VMEM((N_DEV, mc, d_model), jnp.bfloat16),
            pltpu.VMEM((m, d_model), jnp.bfloat16),
            pltpu.SemaphoreType.DMA((N_DEV - 1,)),
            pltpu.SemaphoreType.DMA((N_DEV - 1,)),
            pltpu.SemaphoreType.DMA((N_DEV - 1,)),
            pltpu.SemaphoreType.DMA((N_DEV - 1,)),
        ],
    )(x, W1, W2)


# baseline (device time: 428571 ns/iter reference)
import jax
import jax.numpy as jnp
from jax import lax
from jax.experimental import pallas as pl
from jax.experimental.pallas import tpu as pltpu

N_DEV = 8


def kernel(x, W1, W2):
    m, k_loc = x.shape
    _, d_model = W1.shape
    _, f_loc = W2.shape
    mc = m // N_DEV

    w1b = W1.astype(jnp.bfloat16)
    w2b = W2.astype(jnp.bfloat16)

    def body(x_ref, w1_ref, w2_ref, out_ref,
             xstage, ostage, rs_buf, ag_buf,
             load_sem, store_sem,
             rs_send_sems, rs_recv_sems, ag_send_sems, ag_recv_sems):
        d = lax.axis_index("i")
        right = lax.rem(d + 1, N_DEV)

        def partial_chunk(c):
            cp = pltpu.make_async_copy(
                x_ref.at[pl.ds(c * mc, mc), :], xstage, load_sem)
            cp.start()
            cp.wait()
            xs = xstage[...].astype(jnp.bfloat16)
            return lax.dot(xs, w1_ref[...], preferred_element_type=jnp.float32)

        def store_out(b, block_f32):
            ostage[...] = block_f32
            cp = pltpu.make_async_copy(
                ostage, out_ref.at[pl.ds(b * mc, mc), :], store_sem)
            cp.start()
            cp.wait()

        rs_buf[N_DEV - 1] = partial_chunk(
            lax.rem(d + N_DEV - 1, N_DEV)).astype(jnp.bfloat16)

        h_mine = None
        for s in range(N_DEV - 1):
            send_slot = N_DEV - 1 if s == 0 else s - 1
            rdma = pltpu.make_async_remote_copy(
                src_ref=rs_buf.at[send_slot],
                dst_ref=rs_buf.at[s],
                send_sem=rs_send_sems.at[s],
                recv_sem=rs_recv_sems.at[s],
                device_id=(right,),
                device_id_type=pl.DeviceIdType.MESH,
            )
            rdma.start()
            rdma.wait()
            c = lax.rem(d + 2 * N_DEV - 2 - s, N_DEV)
            p = partial_chunk(c)
            if s < N_DEV - 2:
                rs_buf[s] = (p + rs_buf[s].astype(jnp.float32)).astype(
                    jnp.bfloat16)
            else:
                h_mine = p + rs_buf[s].astype(jnp.float32)

        ag_buf[pl.ds(d * mc, mc), :] = h_mine.astype(jnp.bfloat16)

        store_out(d, lax.dot(h_mine.astype(jnp.bfloat16), w2_ref[...],
                             preferred_element_type=jnp.float32))

        for t in range(N_DEV - 1):
            cs = lax.rem(d + N_DEV - t, N_DEV)
            cr = lax.rem(d + 2 * N_DEV - 1 - t, N_DEV)
            rdma = pltpu.make_async_remote_copy(
                src_ref=ag_buf.at[pl.ds(cs * mc, mc)],
                dst_ref=ag_buf.at[pl.ds(cs * mc, mc)],
                send_sem=ag_send_sems.at[t],
                recv_sem=ag_recv_sems.at[t],
                device_id=(right,),
                device_id_type=pl.DeviceIdType.MESH,
            )
            rdma.start()
            rdma.wait()
            store_out(cr, lax.dot(ag_buf[pl.ds(cr * mc, mc), :], w2_ref[...],
                                  preferred_element_type=jnp.float32))

    return pl.pallas_call(
        body,
        out_shape=jax.ShapeDtypeStruct((m, f_loc), jnp.float32),
        in_specs=[
            pl.BlockSpec(memory_space=pltpu.MemorySpace.HBM),
            pl.BlockSpec(memory_space=pltpu.VMEM),
            pl.BlockSpec(memory_space=pltpu.VMEM),
        ],
        out_specs=pl.BlockSpec(memory_space=pltpu.MemorySpace.HBM),
        scratch_shapes=[
            pltpu.VMEM((mc, k_loc), jnp.float32),
            pltpu.VMEM((mc, f_loc), jnp.float32),
            pltpu.VMEM((N_DEV, mc, d_model), jnp.bfloat16),
            pltpu.VMEM((m, d_model), jnp.bfloat16),
            pltpu.SemaphoreType.DMA,
            pltpu.SemaphoreType.DMA,
            pltpu.SemaphoreType.DMA((N_DEV - 1,)),
            pltpu.SemaphoreType.DMA((N_DEV - 1,)),
            pltpu.SemaphoreType.DMA((N_DEV - 1,)),
            pltpu.SemaphoreType.DMA((N_DEV - 1,)),
        ],
        compiler_params=pltpu.CompilerParams(
            vmem_limit_bytes=60 * 1024 * 1024,
        ),
    )(x, w1b, w2b)


# device time: 244281 ns/iter; 1.7544x vs baseline; 1.7544x over previous
import jax
import jax.numpy as jnp
from jax import lax
from jax.experimental import pallas as pl
from jax.experimental.pallas import tpu as pltpu

N_DEV = 8


def kernel(x, W1, W2):
    m, k_loc = x.shape
    _, d_model = W1.shape
    _, f_loc = W2.shape
    mc = m // N_DEV
    hc = mc // 2

    w1b = W1.astype(jnp.bfloat16)
    w2b = W2.astype(jnp.bfloat16)

    def body(x_ref, w1_ref, w2_ref, out_ref,
             xst, xsb, ostage, rs_cw, rs_ccw, ag_buf,
             lt_sem, lb_sem, st_sem,
             rs_cw_ss, rs_cw_rs, rs_ccw_ss, rs_ccw_rs,
             ag_cw_ss, ag_cw_rs, ag_ccw_ss, ag_ccw_rs):
        d = lax.axis_index("i")
        right = lax.rem(d + 1, N_DEV)
        left = lax.rem(d + N_DEV - 1, N_DEV)

        sends = []

        def half_partials(c_top, c_bot):
            cp1 = pltpu.make_async_copy(
                x_ref.at[pl.ds(c_top * mc, hc), :], xst, lt_sem)
            cp2 = pltpu.make_async_copy(
                x_ref.at[pl.ds(c_bot * mc + hc, hc), :], xsb, lb_sem)
            cp1.start()
            cp2.start()
            cp1.wait()
            pt = lax.dot(xst[...].astype(jnp.bfloat16), w1_ref[...],
                         preferred_element_type=jnp.float32)
            cp2.wait()
            pb = lax.dot(xsb[...].astype(jnp.bfloat16), w1_ref[...],
                         preferred_element_type=jnp.float32)
            return pt, pb

        pt, pb = half_partials(lax.rem(d + N_DEV - 1, N_DEV),
                               lax.rem(d + 1, N_DEV))
        rs_cw[N_DEV - 1] = pt.astype(jnp.bfloat16)
        rs_ccw[N_DEV - 1] = pb.astype(jnp.bfloat16)

        def start_rs(s):
            slot = N_DEV - 1 if s == 0 else s - 1
            r1 = pltpu.make_async_remote_copy(
                src_ref=rs_cw.at[slot], dst_ref=rs_cw.at[s],
                send_sem=rs_cw_ss.at[s], recv_sem=rs_cw_rs.at[s],
                device_id=(right,), device_id_type=pl.DeviceIdType.MESH)
            r2 = pltpu.make_async_remote_copy(
                src_ref=rs_ccw.at[slot], dst_ref=rs_ccw.at[s],
                send_sem=rs_ccw_ss.at[s], recv_sem=rs_ccw_rs.at[s],
                device_id=(left,), device_id_type=pl.DeviceIdType.MESH)
            r1.start()
            r2.start()
            sends.extend((r1, r2))
            return r1, r2

        h_top = h_bot = None
        rs_pairs = [start_rs(0)]
        for s in range(N_DEV - 1):
            pt, pb = half_partials(lax.rem(d + 2 * N_DEV - 2 - s, N_DEV),
                                   lax.rem(d + 2 + s, N_DEV))
            r1, r2 = rs_pairs[s]
            r1.wait_recv()
            r2.wait_recv()
            if s < N_DEV - 2:
                rs_cw[s] = (pt + rs_cw[s].astype(jnp.float32)).astype(
                    jnp.bfloat16)
                rs_ccw[s] = (pb + rs_ccw[s].astype(jnp.float32)).astype(
                    jnp.bfloat16)
                rs_pairs.append(start_rs(s + 1))
            else:
                h_top = pt + rs_cw[s].astype(jnp.float32)
                h_bot = pb + rs_ccw[s].astype(jnp.float32)

        ag_buf[pl.ds(d * mc, hc), :] = h_top.astype(jnp.bfloat16)
        ag_buf[pl.ds(d * mc + hc, hc), :] = h_bot.astype(jnp.bfloat16)

        def out_block(c):
            blk = lax.dot(ag_buf[pl.ds(c * mc, mc), :], w2_ref[...],
                          preferred_element_type=jnp.float32)
            ostage[...] = blk
            cp = pltpu.make_async_copy(
                ostage, out_ref.at[pl.ds(c * mc, mc), :], st_sem)
            cp.start()
            cp.wait()

        def start_ag(t):
            c_cw = lax.rem(d + N_DEV - t, N_DEV)
            c_ccw = lax.rem(d + t, N_DEV)
            a1 = pltpu.make_async_remote_copy(
                src_ref=ag_buf.at[pl.ds(c_cw * mc, hc)],
                dst_ref=ag_buf.at[pl.ds(c_cw * mc, hc)],
                send_sem=ag_cw_ss.at[t], recv_sem=ag_cw_rs.at[t],
                device_id=(right,), device_id_type=pl.DeviceIdType.MESH)
            a2 = pltpu.make_async_remote_copy(
                src_ref=ag_buf.at[pl.ds(c_ccw * mc + hc, hc)],
                dst_ref=ag_buf.at[pl.ds(c_ccw * mc + hc, hc)],
                send_sem=ag_ccw_ss.at[t], recv_sem=ag_ccw_rs.at[t],
                device_id=(left,), device_id_type=pl.DeviceIdType.MESH)
            a1.start()
            a2.start()
            sends.extend((a1, a2))
            return a1, a2

        ag_pairs = [start_ag(0)]
        out_block(d)
        for t in range(N_DEV - 1):
            a1, a2 = ag_pairs[t]
            a1.wait_recv()
            a2.wait_recv()
            if t < N_DEV - 2:
                ag_pairs.append(start_ag(t + 1))
            if t == 3:
                out_block(lax.rem(d + 4, N_DEV))
            elif t > 3:
                out_block(lax.rem(d + N_DEV - 1 - t, N_DEV))
                out_block(lax.rem(d + 1 + t, N_DEV))

        for r in sends:
            r.wait_send()

    return pl.pallas_call(
        body,
        out_shape=jax.ShapeDtypeStruct((m, f_loc), jnp.float32),
        in_specs=[
            pl.BlockSpec(memory_space=pltpu.MemorySpace.HBM),
            pl.BlockSpec(memory_space=pltpu.MemorySpace.VMEM),
            pl.BlockSpec(memory_space=pltpu.MemorySpace.VMEM),
        ],
        out_specs=pl.BlockSpec(memory_space=pltpu.MemorySpace.HBM),
        scratch_shapes=[
            pltpu.VMEM((hc, k_loc), jnp.float32),
            pltpu.VMEM((hc, k_loc), jnp.float32),
            pltpu.VMEM((mc, f_loc), jnp.float32),
            pltpu.VMEM((N_DEV, hc, d_model), jnp.bfloat16),
            pltpu.VMEM((N_DEV, hc, d_model), jnp.bfloat16),
            pltpu.VMEM((m, d_model), jnp.bfloat16),
            pltpu.SemaphoreType.DMA,
            pltpu.SemaphoreType.DMA,
            pltpu.SemaphoreType.DMA,
            pltpu.SemaphoreType.DMA((N_DEV - 1,)),
            pltpu.SemaphoreType.DMA((N_DEV - 1,)),
            pltpu.SemaphoreType.DMA((N_DEV - 1,)),
            pltpu.SemaphoreType.DMA((N_DEV - 1,)),
            pltpu.SemaphoreType.DMA((N_DEV - 1,)),
            pltpu.SemaphoreType.DMA((N_DEV - 1,)),
            pltpu.SemaphoreType.DMA((N_DEV - 1,)),
            pltpu.SemaphoreType.DMA((N_DEV - 1,)),
        ],
        compiler_params=pltpu.CompilerParams(
            vmem_limit_bytes=60 * 1024 * 1024,
        ),
    )(x, w1b, w2b)


# device time: 231654 ns/iter; 1.8500x vs baseline; 1.0545x over previous
import jax
import jax.numpy as jnp
from jax import lax
from jax.experimental import pallas as pl
from jax.experimental.pallas import tpu as pltpu

N_DEV = 8


def kernel(x, W1, W2):
    m, k_loc = x.shape
    _, d_model = W1.shape
    _, f_loc = W2.shape
    mc = m // N_DEV
    hc = mc // 2

    w1b = W1.astype(jnp.bfloat16)
    w2b = W2.astype(jnp.bfloat16)

    def body(x_ref, w1_ref, w2_ref, out_ref,
             xst, xsb, ostage, rs_cw, rs_ccw, ag_buf,
             lt_sem, lb_sem, st_sems,
             rs_cw_ss, rs_cw_rs, rs_ccw_ss, rs_ccw_rs,
             ag_cw_ss, ag_cw_rs, ag_ccw_ss, ag_ccw_rs):
        d = lax.axis_index("i")
        right = lax.rem(d + 1, N_DEV)
        left = lax.rem(d + N_DEV - 1, N_DEV)

        sends = []

        def half_partials(c_top, c_bot):
            cp1 = pltpu.make_async_copy(
                x_ref.at[pl.ds(c_top * mc, hc), :], xst, lt_sem)
            cp2 = pltpu.make_async_copy(
                x_ref.at[pl.ds(c_bot * mc + hc, hc), :], xsb, lb_sem)
            cp1.start()
            cp2.start()
            cp1.wait()
            pt = lax.dot(xst[...].astype(jnp.bfloat16), w1_ref[...],
                         preferred_element_type=jnp.float32)
            cp2.wait()
            pb = lax.dot(xsb[...].astype(jnp.bfloat16), w1_ref[...],
                         preferred_element_type=jnp.float32)
            return pt, pb

        pt, pb = half_partials(lax.rem(d + N_DEV - 1, N_DEV),
                               lax.rem(d + 1, N_DEV))
        rs_cw[N_DEV - 1] = pt.astype(jnp.bfloat16)
        rs_ccw[N_DEV - 1] = pb.astype(jnp.bfloat16)

        def start_rs_cw(s):
            slot = N_DEV - 1 if s == 0 else s - 1
            r = pltpu.make_async_remote_copy(
                src_ref=rs_cw.at[slot], dst_ref=rs_cw.at[s],
                send_sem=rs_cw_ss.at[s], recv_sem=rs_cw_rs.at[s],
                device_id=(right,), device_id_type=pl.DeviceIdType.MESH)
            r.start()
            sends.append(r)
            return r

        def start_rs_ccw(s):
            slot = N_DEV - 1 if s == 0 else s - 1
            r = pltpu.make_async_remote_copy(
                src_ref=rs_ccw.at[slot], dst_ref=rs_ccw.at[s],
                send_sem=rs_ccw_ss.at[s], recv_sem=rs_ccw_rs.at[s],
                device_id=(left,), device_id_type=pl.DeviceIdType.MESH)
            r.start()
            sends.append(r)
            return r

        h_top = h_bot = None
        r1 = start_rs_cw(0)
        r2 = start_rs_ccw(0)
        for s in range(N_DEV - 1):
            pt, pb = half_partials(lax.rem(d + 2 * N_DEV - 2 - s, N_DEV),
                                   lax.rem(d + 2 + s, N_DEV))
            r1.wait_recv()
            if s < N_DEV - 2:
                rs_cw[s] = (pt + rs_cw[s].astype(jnp.float32)).astype(
                    jnp.bfloat16)
                r1 = start_rs_cw(s + 1)
            else:
                h_top = pt + rs_cw[s].astype(jnp.float32)
            r2.wait_recv()
            if s < N_DEV - 2:
                rs_ccw[s] = (pb + rs_ccw[s].astype(jnp.float32)).astype(
                    jnp.bfloat16)
                r2 = start_rs_ccw(s + 1)
            else:
                h_bot = pb + rs_ccw[s].astype(jnp.float32)

        ag_buf[pl.ds(d * mc, hc), :] = h_top.astype(jnp.bfloat16)
        ag_buf[pl.ds(d * mc + hc, hc), :] = h_bot.astype(jnp.bfloat16)

        prev_store = [None, None]

        def store_half(c, top):
            slot = 0 if top else 1
            off = 0 if top else hc
            if prev_store[slot] is not None:
                prev_store[slot].wait()
            ostage[slot] = lax.dot(
                ag_buf[pl.ds(c * mc + off, hc), :], w2_ref[...],
                preferred_element_type=jnp.float32)
            cp = pltpu.make_async_copy(
                ostage.at[slot], out_ref.at[pl.ds(c * mc + off, hc), :],
                st_sems.at[slot])
            cp.start()
            prev_store[slot] = cp

        def start_ag_cw(t):
            c = lax.rem(d + N_DEV - t, N_DEV)
            a = pltpu.make_async_remote_copy(
                src_ref=ag_buf.at[pl.ds(c * mc, hc)],
                dst_ref=ag_buf.at[pl.ds(c * mc, hc)],
                send_sem=ag_cw_ss.at[t], recv_sem=ag_cw_rs.at[t],
                device_id=(right,), device_id_type=pl.DeviceIdType.MESH)
            a.start()
            sends.append(a)
            return a

        def start_ag_ccw(t):
            c = lax.rem(d + t, N_DEV)
            a = pltpu.make_async_remote_copy(
                src_ref=ag_buf.at[pl.ds(c * mc + hc, hc)],
                dst_ref=ag_buf.at[pl.ds(c * mc + hc, hc)],
                send_sem=ag_ccw_ss.at[t], recv_sem=ag_ccw_rs.at[t],
                device_id=(left,), device_id_type=pl.DeviceIdType.MESH)
            a.start()
            sends.append(a)
            return a

        a1 = start_ag_cw(0)
        a2 = start_ag_ccw(0)
        store_half(d, top=True)
        store_half(d, top=False)
        for t in range(N_DEV - 1):
            a1.wait_recv()
            if t < N_DEV - 2:
                a1 = start_ag_cw(t + 1)
            store_half(lax.rem(d + 2 * N_DEV - 1 - t, N_DEV), top=True)
            a2.wait_recv()
            if t < N_DEV - 2:
                a2 = start_ag_ccw(t + 1)
            store_half(lax.rem(d + 1 + t, N_DEV), top=False)

        prev_store[0].wait()
        prev_store[1].wait()
        for r in sends:
            r.wait_send()

    return pl.pallas_call(
        body,
        out_shape=jax.ShapeDtypeStruct((m, f_loc), jnp.float32),
        in_specs=[
            pl.BlockSpec(memory_space=pltpu.MemorySpace.HBM),
            pl.BlockSpec(memory_space=pltpu.MemorySpace.VMEM),
            pl.BlockSpec(memory_space=pltpu.MemorySpace.VMEM),
        ],
        out_specs=pl.BlockSpec(memory_space=pltpu.MemorySpace.HBM),
        scratch_shapes=[
            pltpu.VMEM((hc, k_loc), jnp.float32),
            pltpu.VMEM((hc, k_loc), jnp.float32),
            pltpu.VMEM((2, hc, f_loc), jnp.float32),
            pltpu.VMEM((N_DEV, hc, d_model), jnp.bfloat16),
            pltpu.VMEM((N_DEV, hc, d_model), jnp.bfloat16),
            pltpu.VMEM((m, d_model), jnp.bfloat16),
            pltpu.SemaphoreType.DMA,
            pltpu.SemaphoreType.DMA,
            pltpu.SemaphoreType.DMA((2,)),
            pltpu.SemaphoreType.DMA((N_DEV - 1,)),
            pltpu.SemaphoreType.DMA((N_DEV - 1,)),
            pltpu.SemaphoreType.DMA((N_DEV - 1,)),
            pltpu.SemaphoreType.DMA((N_DEV - 1,)),
            pltpu.SemaphoreType.DMA((N_DEV - 1,)),
            pltpu.SemaphoreType.DMA((N_DEV - 1,)),
            pltpu.SemaphoreType.DMA((N_DEV - 1,)),
            pltpu.SemaphoreType.DMA((N_DEV - 1,)),
        ],
        compiler_params=pltpu.CompilerParams(
            vmem_limit_bytes=60 * 1024 * 1024,
        ),
    )(x, w1b, w2b)


# device time: 201218 ns/iter; 2.1299x vs baseline; 1.1513x over previous
import jax
import jax.numpy as jnp
from jax import lax
from jax.experimental import pallas as pl
from jax.experimental.pallas import tpu as pltpu

N_DEV = 8


def kernel(x, W1, W2):
    m, k_loc = x.shape
    _, d_model = W1.shape
    _, f_loc = W2.shape
    mc = m // N_DEV
    hc = mc // 2
    qc = hc // 2

    w1b = W1.astype(jnp.bfloat16)
    w2b = W2.astype(jnp.bfloat16)

    def body(x_ref, w1_ref, w2_ref, out_ref,
             xst, xsb, ostage, rs_cw, rs_ccw, ag_buf,
             lt_sem, lb_sem, st_sems,
             rs_cw_ss, rs_cw_rs, rs_ccw_ss, rs_ccw_rs,
             ag_cw_ss, ag_cw_rs, ag_ccw_ss, ag_ccw_rs):
        d = lax.axis_index("i")
        right = lax.rem(d + 1, N_DEV)
        left = lax.rem(d + N_DEV - 1, N_DEV)

        sends = []

        def half_partials(c_top, c_bot):
            cp1 = pltpu.make_async_copy(
                x_ref.at[pl.ds(c_top * mc, hc), :], xst, lt_sem)
            cp2 = pltpu.make_async_copy(
                x_ref.at[pl.ds(c_bot * mc + hc, hc), :], xsb, lb_sem)
            cp1.start()
            cp2.start()
            cp1.wait()
            pt = lax.dot(xst[...].astype(jnp.bfloat16), w1_ref[...],
                         preferred_element_type=jnp.float32)
            cp2.wait()
            pb = lax.dot(xsb[...].astype(jnp.bfloat16), w1_ref[...],
                         preferred_element_type=jnp.float32)
            return pt, pb

        pt, pb = half_partials(lax.rem(d + N_DEV - 1, N_DEV),
                               lax.rem(d + 1, N_DEV))
        rs_cw[N_DEV - 1] = pt.astype(jnp.bfloat16)
        rs_ccw[N_DEV - 1] = pb.astype(jnp.bfloat16)

        def start_rs_cw(s, q):
            slot = N_DEV - 1 if s == 0 else s - 1
            r = pltpu.make_async_remote_copy(
                src_ref=rs_cw.at[slot, pl.ds(q * qc, qc)],
                dst_ref=rs_cw.at[s, pl.ds(q * qc, qc)],
                send_sem=rs_cw_ss.at[s, q], recv_sem=rs_cw_rs.at[s, q],
                device_id=(right,), device_id_type=pl.DeviceIdType.MESH)
            r.start()
            sends.append(r)
            return r

        def start_rs_ccw(s, q):
            slot = N_DEV - 1 if s == 0 else s - 1
            r = pltpu.make_async_remote_copy(
                src_ref=rs_ccw.at[slot, pl.ds(q * qc, qc)],
                dst_ref=rs_ccw.at[s, pl.ds(q * qc, qc)],
                send_sem=rs_ccw_ss.at[s, q], recv_sem=rs_ccw_rs.at[s, q],
                device_id=(left,), device_id_type=pl.DeviceIdType.MESH)
            r.start()
            sends.append(r)
            return r

        h_top = h_bot = None
        rcw = [start_rs_cw(0, 0), start_rs_cw(0, 1)]
        rccw = [start_rs_ccw(0, 0), start_rs_ccw(0, 1)]
        for s in range(N_DEV - 1):
            pt, pb = half_partials(lax.rem(d + 2 * N_DEV - 2 - s, N_DEV),
                                   lax.rem(d + 2 + s, N_DEV))
            if s < N_DEV - 2:
                nrcw, nrccw = [None, None], [None, None]
                for q in range(2):
                    sl = pl.ds(q * qc, qc)
                    rcw[q].wait_recv()
                    rs_cw[s, sl] = (
                        pt[q * qc:(q + 1) * qc]
                        + rs_cw[s, sl].astype(jnp.float32)).astype(
                            jnp.bfloat16)
                    nrcw[q] = start_rs_cw(s + 1, q)
                    rccw[q].wait_recv()
                    rs_ccw[s, sl] = (
                        pb[q * qc:(q + 1) * qc]
                        + rs_ccw[s, sl].astype(jnp.float32)).astype(
                            jnp.bfloat16)
                    nrccw[q] = start_rs_ccw(s + 1, q)
                rcw, rccw = nrcw, nrccw
            else:
                rcw[0].wait_recv()
                rcw[1].wait_recv()
                h_top = pt + rs_cw[s].astype(jnp.float32)
                rccw[0].wait_recv()
                rccw[1].wait_recv()
                h_bot = pb + rs_ccw[s].astype(jnp.float32)

        ag_buf[pl.ds(d * mc, hc), :] = h_top.astype(jnp.bfloat16)
        ag_buf[pl.ds(d * mc + hc, hc), :] = h_bot.astype(jnp.bfloat16)

        prev_store = [None, None]

        def store_half(c, top):
            slot = 0 if top else 1
            off = 0 if top else hc
            if prev_store[slot] is not None:
                prev_store[slot].wait()
            ostage[slot] = lax.dot(
                ag_buf[pl.ds(c * mc + off, hc), :], w2_ref[...],
                preferred_element_type=jnp.float32)
            cp = pltpu.make_async_copy(
                ostage.at[slot], out_ref.at[pl.ds(c * mc + off, hc), :],
                st_sems.at[slot])
            cp.start()
            prev_store[slot] = cp

        def start_ag_cw(t, q):
            c = lax.rem(d + N_DEV - t, N_DEV)
            a = pltpu.make_async_remote_copy(
                src_ref=ag_buf.at[pl.ds(c * mc + q * qc, qc)],
                dst_ref=ag_buf.at[pl.ds(c * mc + q * qc, qc)],
                send_sem=ag_cw_ss.at[t, q], recv_sem=ag_cw_rs.at[t, q],
                device_id=(right,), device_id_type=pl.DeviceIdType.MESH)
            a.start()
            sends.append(a)
            return a

        def start_ag_ccw(t, q):
            c = lax.rem(d + t, N_DEV)
            a = pltpu.make_async_remote_copy(
                src_ref=ag_buf.at[pl.ds(c * mc + hc + q * qc, qc)],
                dst_ref=ag_buf.at[pl.ds(c * mc + hc + q * qc, qc)],
                send_sem=ag_ccw_ss.at[t, q], recv_sem=ag_ccw_rs.at[t, q],
                device_id=(left,), device_id_type=pl.DeviceIdType.MESH)
            a.start()
            sends.append(a)
            return a

        acw = [start_ag_cw(0, 0), start_ag_cw(0, 1)]
        accw = [start_ag_ccw(0, 0), start_ag_ccw(0, 1)]
        store_half(d, top=True)
        store_half(d, top=False)
        for t in range(N_DEV - 1):
            nacw, naccw = [None, None], [None, None]
            for q in range(2):
                acw[q].wait_recv()
                if t < N_DEV - 2:
                    nacw[q] = start_ag_cw(t + 1, q)
                accw[q].wait_recv()
                if t < N_DEV - 2:
                    naccw[q] = start_ag_ccw(t + 1, q)
            acw, accw = nacw, naccw
            store_half(lax.rem(d + 2 * N_DEV - 1 - t, N_DEV), top=True)
            store_half(lax.rem(d + 1 + t, N_DEV), top=False)

        prev_store[0].wait()
        prev_store[1].wait()
        for r in sends:
            r.wait_send()

    return pl.pallas_call(
        body,
        out_shape=jax.ShapeDtypeStruct((m, f_loc), jnp.float32),
        in_specs=[
            pl.BlockSpec(memory_space=pltpu.MemorySpace.HBM),
            pl.BlockSpec(memory_space=pltpu.MemorySpace.VMEM),
            pl.BlockSpec(memory_space=pltpu.MemorySpace.VMEM),
        ],
        out_specs=pl.BlockSpec(memory_space=pltpu.MemorySpace.HBM),
        scratch_shapes=[
            pltpu.VMEM((hc, k_loc), jnp.float32),
            pltpu.VMEM((hc, k_loc), jnp.float32),
            pltpu.VMEM((2, hc, f_loc), jnp.float32),
            pltpu.VMEM((N_DEV, hc, d_model), jnp.bfloat16),
            pltpu.VMEM((N_DEV, hc, d_model), jnp.bfloat16),
            pltpu.VMEM((m, d_model), jnp.bfloat16),
            pltpu.SemaphoreType.DMA,
            pltpu.SemaphoreType.DMA,
            pltpu.SemaphoreType.DMA((2,)),
            pltpu.SemaphoreType.DMA((N_DEV - 1, 2)),
            pltpu.SemaphoreType.DMA((N_DEV - 1, 2)),
            pltpu.SemaphoreType.DMA((N_DEV - 1, 2)),
            pltpu.SemaphoreType.DMA((N_DEV - 1, 2)),
            pltpu.SemaphoreType.DMA((N_DEV - 1, 2)),
            pltpu.SemaphoreType.DMA((N_DEV - 1, 2)),
            pltpu.SemaphoreType.DMA((N_DEV - 1, 2)),
            pltpu.SemaphoreType.DMA((N_DEV - 1, 2)),
        ],
        compiler_params=pltpu.CompilerParams(
            vmem_limit_bytes=60 * 1024 * 1024,
        ),
    )(x, w1b, w2b)


# device time: 193962 ns/iter; 2.2096x vs baseline; 1.0374x over previous
import jax
import jax.numpy as jnp
from jax import lax
from jax.experimental import pallas as pl
from jax.experimental.pallas import tpu as pltpu

N_DEV = 8


def kernel(x, W1, W2):
    m, k_loc = x.shape
    _, d_model = W1.shape
    _, f_loc = W2.shape
    mc = m // N_DEV
    hc = mc // 2
    qc = hc // 2

    w1b = W1.astype(jnp.bfloat16)
    w2b = W2.astype(jnp.bfloat16)

    def body(x_ref, w1_ref, w2_ref, out_ref,
             xst, xsb, ostage, rs_cw, rs_ccw, ag_buf,
             lt_sem, lb_sem, st_sems,
             rs_cw_ss, rs_cw_rs, rs_ccw_ss, rs_ccw_rs,
             ag_cw_ss, ag_cw_rs, ag_ccw_ss, ag_ccw_rs):
        d = lax.axis_index("i")
        right = lax.rem(d + 1, N_DEV)
        left = lax.rem(d + N_DEV - 1, N_DEV)

        sends = []

        def start_loads(c_top, c_bot):
            cp1 = pltpu.make_async_copy(
                x_ref.at[pl.ds(c_top * mc, hc), :], xst, lt_sem)
            cp2 = pltpu.make_async_copy(
                x_ref.at[pl.ds(c_bot * mc + hc, hc), :], xsb, lb_sem)
            cp1.start()
            cp2.start()
            return cp1, cp2

        def qdot(stage, q):
            return lax.dot(
                stage[pl.ds(q * qc, qc), :].astype(jnp.bfloat16),
                w1_ref[...], preferred_element_type=jnp.float32)

        def start_rs_cw(s, q):
            slot = N_DEV - 1 if s == 0 else s - 1
            r = pltpu.make_async_remote_copy(
                src_ref=rs_cw.at[slot, pl.ds(q * qc, qc)],
                dst_ref=rs_cw.at[s, pl.ds(q * qc, qc)],
                send_sem=rs_cw_ss.at[s, q], recv_sem=rs_cw_rs.at[s, q],
                device_id=(right,), device_id_type=pl.DeviceIdType.MESH)
            r.start()
            sends.append(r)
            return r

        def start_rs_ccw(s, q):
            slot = N_DEV - 1 if s == 0 else s - 1
            r = pltpu.make_async_remote_copy(
                src_ref=rs_ccw.at[slot, pl.ds(q * qc, qc)],
                dst_ref=rs_ccw.at[s, pl.ds(q * qc, qc)],
                send_sem=rs_ccw_ss.at[s, q], recv_sem=rs_ccw_rs.at[s, q],
                device_id=(left,), device_id_type=pl.DeviceIdType.MESH)
            r.start()
            sends.append(r)
            return r

        def start_ag_cw(t, q):
            c = lax.rem(d + N_DEV - t, N_DEV)
            a = pltpu.make_async_remote_copy(
                src_ref=ag_buf.at[pl.ds(c * mc + q * qc, qc)],
                dst_ref=ag_buf.at[pl.ds(c * mc + q * qc, qc)],
                send_sem=ag_cw_ss.at[t, q], recv_sem=ag_cw_rs.at[t, q],
                device_id=(right,), device_id_type=pl.DeviceIdType.MESH)
            a.start()
            sends.append(a)
            return a

        def start_ag_ccw(t, q):
            c = lax.rem(d + t, N_DEV)
            a = pltpu.make_async_remote_copy(
                src_ref=ag_buf.at[pl.ds(c * mc + hc + q * qc, qc)],
                dst_ref=ag_buf.at[pl.ds(c * mc + hc + q * qc, qc)],
                send_sem=ag_ccw_ss.at[t, q], recv_sem=ag_ccw_rs.at[t, q],
                device_id=(left,), device_id_type=pl.DeviceIdType.MESH)
            a.start()
            sends.append(a)
            return a

        prev_store = [None, None]

        def store_half(c, top):
            slot = 0 if top else 1
            off = 0 if top else hc
            if prev_store[slot] is not None:
                prev_store[slot].wait()
            ostage[slot] = lax.dot(
                ag_buf[pl.ds(c * mc + off, hc), :], w2_ref[...],
                preferred_element_type=jnp.float32)
            cp = pltpu.make_async_copy(
                ostage.at[slot], out_ref.at[pl.ds(c * mc + off, hc), :],
                st_sems.at[slot])
            cp.start()
            prev_store[slot] = cp

        cp1, cp2 = start_loads(lax.rem(d + N_DEV - 1, N_DEV),
                               lax.rem(d + 1, N_DEV))
        cp1.wait()
        rs_cw[N_DEV - 1, pl.ds(0, qc)] = qdot(xst, 0).astype(jnp.bfloat16)
        rcw = [start_rs_cw(0, 0), None]
        cp2.wait()
        rs_ccw[N_DEV - 1, pl.ds(0, qc)] = qdot(xsb, 0).astype(jnp.bfloat16)
        rccw = [start_rs_ccw(0, 0), None]
        rs_cw[N_DEV - 1, pl.ds(qc, qc)] = qdot(xst, 1).astype(jnp.bfloat16)
        rcw[1] = start_rs_cw(0, 1)
        rs_ccw[N_DEV - 1, pl.ds(qc, qc)] = qdot(xsb, 1).astype(jnp.bfloat16)
        rccw[1] = start_rs_ccw(0, 1)

        q0, q1 = pl.ds(0, qc), pl.ds(qc, qc)
        acw = [None, None]
        accw = [None, None]
        for s in range(N_DEV - 1):
            cp1, cp2 = start_loads(lax.rem(d + 2 * N_DEV - 2 - s, N_DEV),
                                   lax.rem(d + 2 + s, N_DEV))
            cp1.wait()
            pt0 = qdot(xst, 0)
            cp2.wait()
            pb0 = qdot(xsb, 0)
            last = s == N_DEV - 2
            rcw[0].wait_recv()
            acc = pt0 + rs_cw[s, q0].astype(jnp.float32)
            if not last:
                rs_cw[s, q0] = acc.astype(jnp.bfloat16)
                ncw0 = start_rs_cw(s + 1, 0)
            else:
                ag_buf[pl.ds(d * mc, qc), :] = acc.astype(jnp.bfloat16)
                acw[0] = start_ag_cw(0, 0)
            rccw[0].wait_recv()
            acc = pb0 + rs_ccw[s, q0].astype(jnp.float32)
            if not last:
                rs_ccw[s, q0] = acc.astype(jnp.bfloat16)
                nccw0 = start_rs_ccw(s + 1, 0)
            else:
                ag_buf[pl.ds(d * mc + hc, qc), :] = acc.astype(jnp.bfloat16)
                accw[0] = start_ag_ccw(0, 0)
            pt1 = qdot(xst, 1)
            pb1 = qdot(xsb, 1)
            rcw[1].wait_recv()
            acc = pt1 + rs_cw[s, q1].astype(jnp.float32)
            if not last:
                rs_cw[s, q1] = acc.astype(jnp.bfloat16)
                rcw = [ncw0, start_rs_cw(s + 1, 1)]
            else:
                ag_buf[pl.ds(d * mc + qc, qc), :] = acc.astype(jnp.bfloat16)
                acw[1] = start_ag_cw(0, 1)
            rccw[1].wait_recv()
            acc = pb1 + rs_ccw[s, q1].astype(jnp.float32)
            if not last:
                rs_ccw[s, q1] = acc.astype(jnp.bfloat16)
                rccw = [nccw0, start_rs_ccw(s + 1, 1)]
            else:
                ag_buf[pl.ds(d * mc + hc + qc, qc), :] = acc.astype(
                    jnp.bfloat16)
                accw[1] = start_ag_ccw(0, 1)

        store_half(d, top=True)
        store_half(d, top=False)
        for t in range(N_DEV - 1):
            nacw, naccw = [None, None], [None, None]
            for q in range(2):
                acw[q].wait_recv()
                if t < N_DEV - 2:
                    nacw[q] = start_ag_cw(t + 1, q)
                accw[q].wait_recv()
                if t < N_DEV - 2:
                    naccw[q] = start_ag_ccw(t + 1, q)
            acw, accw = nacw, naccw
            store_half(lax.rem(d + 2 * N_DEV - 1 - t, N_DEV), top=True)
            store_half(lax.rem(d + 1 + t, N_DEV), top=False)

        prev_store[0].wait()
        prev_store[1].wait()
        for r in sends:
            r.wait_send()

    return pl.pallas_call(
        body,
        out_shape=jax.ShapeDtypeStruct((m, f_loc), jnp.float32),
        in_specs=[
            pl.BlockSpec(memory_space=pltpu.MemorySpace.HBM),
            pl.BlockSpec(memory_space=pltpu.MemorySpace.VMEM),
            pl.BlockSpec(memory_space=pltpu.MemorySpace.VMEM),
        ],
        out_specs=pl.BlockSpec(memory_space=pltpu.MemorySpace.HBM),
        scratch_shapes=[
            pltpu.VMEM((hc, k_loc), jnp.float32),
            pltpu.VMEM((hc, k_loc), jnp.float32),
            pltpu.VMEM((2, hc, f_loc), jnp.float32),
            pltpu.VMEM((N_DEV, hc, d_model), jnp.bfloat16),
            pltpu.VMEM((N_DEV, hc, d_model), jnp.bfloat16),
            pltpu.VMEM((m, d_model), jnp.bfloat16),
            pltpu.SemaphoreType.DMA,
            pltpu.SemaphoreType.DMA,
            pltpu.SemaphoreType.DMA((2,)),
            pltpu.SemaphoreType.DMA((N_DEV - 1, 2)),
            pltpu.SemaphoreType.DMA((N_DEV - 1, 2)),
            pltpu.SemaphoreType.DMA((N_DEV - 1, 2)),
            pltpu.SemaphoreType.DMA((N_DEV - 1, 2)),
            pltpu.SemaphoreType.DMA((N_DEV - 1, 2)),
            pltpu.SemaphoreType.DMA((N_DEV - 1, 2)),
            pltpu.SemaphoreType.DMA((N_DEV - 1, 2)),
            pltpu.SemaphoreType.DMA((N_DEV - 1, 2)),
        ],
        compiler_params=pltpu.CompilerParams(
            vmem_limit_bytes=60 * 1024 * 1024,
        ),
    )(x, w1b, w2b)


# device time: 189050 ns/iter; 2.2670x vs baseline; 1.0260x over previous
import jax
import jax.numpy as jnp
from jax import lax
from jax.experimental import pallas as pl
from jax.experimental.pallas import tpu as pltpu

N_DEV = 8


def kernel(x, W1, W2):
    m, k_loc = x.shape
    _, d_model = W1.shape
    _, f_loc = W2.shape
    mc = m // N_DEV
    hc = mc // 2
    qc = hc // 2

    w1b = W1.astype(jnp.bfloat16)
    w2b = W2.astype(jnp.bfloat16)

    def body(x_ref, w1_ref, w2_ref, out_ref,
             xst, xsb, ostage, rs_cw, rs_ccw, ag_buf,
             lt_sem, lb_sem, st_sems,
             rs_cw_ss, rs_cw_rs, rs_ccw_ss, rs_ccw_rs,
             ag_cw_ss, ag_cw_rs, ag_ccw_ss, ag_ccw_rs):
        d = lax.axis_index("i")
        right = lax.rem(d + 1, N_DEV)
        left = lax.rem(d + N_DEV - 1, N_DEV)

        sends = []

        def start_loads(c_top, c_bot):
            cp1 = pltpu.make_async_copy(
                x_ref.at[pl.ds(c_top * mc, hc), :], xst, lt_sem)
            cp2 = pltpu.make_async_copy(
                x_ref.at[pl.ds(c_bot * mc + hc, hc), :], xsb, lb_sem)
            cp1.start()
            cp2.start()
            return cp1, cp2

        def qdot(stage, q):
            return lax.dot(
                stage[pl.ds(q * qc, qc), :].astype(jnp.bfloat16),
                w1_ref[...], preferred_element_type=jnp.float32)

        def start_rs_cw(s, q):
            slot = N_DEV - 1 if s == 0 else s - 1
            r = pltpu.make_async_remote_copy(
                src_ref=rs_cw.at[slot, pl.ds(q * qc, qc)],
                dst_ref=rs_cw.at[s, pl.ds(q * qc, qc)],
                send_sem=rs_cw_ss.at[s, q], recv_sem=rs_cw_rs.at[s, q],
                device_id=(right,), device_id_type=pl.DeviceIdType.MESH)
            r.start()
            sends.append(r)
            return r

        def start_rs_ccw(s, q):
            slot = N_DEV - 1 if s == 0 else s - 1
            r = pltpu.make_async_remote_copy(
                src_ref=rs_ccw.at[slot, pl.ds(q * qc, qc)],
                dst_ref=rs_ccw.at[s, pl.ds(q * qc, qc)],
                send_sem=rs_ccw_ss.at[s, q], recv_sem=rs_ccw_rs.at[s, q],
                device_id=(left,), device_id_type=pl.DeviceIdType.MESH)
            r.start()
            sends.append(r)
            return r

        def start_ag_cw(t, q):
            c = lax.rem(d + N_DEV - t, N_DEV)
            a = pltpu.make_async_remote_copy(
                src_ref=ag_buf.at[pl.ds(c * mc + q * qc, qc)],
                dst_ref=ag_buf.at[pl.ds(c * mc + q * qc, qc)],
                send_sem=ag_cw_ss.at[t, q], recv_sem=ag_cw_rs.at[t, q],
                device_id=(right,), device_id_type=pl.DeviceIdType.MESH)
            a.start()
            sends.append(a)
            return a

        def start_ag_ccw(t, q):
            c = lax.rem(d + t, N_DEV)
            a = pltpu.make_async_remote_copy(
                src_ref=ag_buf.at[pl.ds(c * mc + hc + q * qc, qc)],
                dst_ref=ag_buf.at[pl.ds(c * mc + hc + q * qc, qc)],
                send_sem=ag_ccw_ss.at[t, q], recv_sem=ag_ccw_rs.at[t, q],
                device_id=(left,), device_id_type=pl.DeviceIdType.MESH)
            a.start()
            sends.append(a)
            return a

        prev_store = [None, None]

        def store_half(c, top):
            slot = 0 if top else 1
            off = 0 if top else hc
            if prev_store[slot] is not None:
                prev_store[slot].wait()
            ostage[slot] = lax.dot(
                ag_buf[pl.ds(c * mc + off, hc), :], w2_ref[...],
                preferred_element_type=jnp.float32)
            cp = pltpu.make_async_copy(
                ostage.at[slot], out_ref.at[pl.ds(c * mc + off, hc), :],
                st_sems.at[slot])
            cp.start()
            prev_store[slot] = cp

        cp1, cp2 = start_loads(lax.rem(d + N_DEV - 1, N_DEV),
                               lax.rem(d + 1, N_DEV))

        barrier_sem = pltpu.get_barrier_semaphore()
        pl.semaphore_signal(barrier_sem, inc=1, device_id=(left,),
                            device_id_type=pl.DeviceIdType.MESH)
        pl.semaphore_signal(barrier_sem, inc=1, device_id=(right,),
                            device_id_type=pl.DeviceIdType.MESH)
        pl.semaphore_wait(barrier_sem, 2)

        cp1.wait()
        rs_cw[N_DEV - 1, pl.ds(0, qc)] = qdot(xst, 0).astype(jnp.bfloat16)
        rcw = [start_rs_cw(0, 0), None]
        cp2.wait()
        rs_ccw[N_DEV - 1, pl.ds(0, qc)] = qdot(xsb, 0).astype(jnp.bfloat16)
        rccw = [start_rs_ccw(0, 0), None]
        rs_cw[N_DEV - 1, pl.ds(qc, qc)] = qdot(xst, 1).astype(jnp.bfloat16)
        rcw[1] = start_rs_cw(0, 1)
        rs_ccw[N_DEV - 1, pl.ds(qc, qc)] = qdot(xsb, 1).astype(jnp.bfloat16)
        rccw[1] = start_rs_ccw(0, 1)

        q0, q1 = pl.ds(0, qc), pl.ds(qc, qc)
        acw = [None, None]
        accw = [None, None]
        for s in range(N_DEV - 1):
            cp1, cp2 = start_loads(lax.rem(d + 2 * N_DEV - 2 - s, N_DEV),
                                   lax.rem(d + 2 + s, N_DEV))
            cp1.wait()
            pt0 = qdot(xst, 0)
            cp2.wait()
            pb0 = qdot(xsb, 0)
            last = s == N_DEV - 2
            rcw[0].wait_recv()
            acc = pt0 + rs_cw[s, q0].astype(jnp.float32)
            if not last:
                rs_cw[s, q0] = acc.astype(jnp.bfloat16)
                ncw0 = start_rs_cw(s + 1, 0)
            else:
                ag_buf[pl.ds(d * mc, qc), :] = acc.astype(jnp.bfloat16)
                acw[0] = start_ag_cw(0, 0)
            rccw[0].wait_recv()
            acc = pb0 + rs_ccw[s, q0].astype(jnp.float32)
            if not last:
                rs_ccw[s, q0] = acc.astype(jnp.bfloat16)
                nccw0 = start_rs_ccw(s + 1, 0)
            else:
                ag_buf[pl.ds(d * mc + hc, qc), :] = acc.astype(jnp.bfloat16)
                accw[0] = start_ag_ccw(0, 0)
            pt1 = qdot(xst, 1)
            pb1 = qdot(xsb, 1)
            rcw[1].wait_recv()
            acc = pt1 + rs_cw[s, q1].astype(jnp.float32)
            if not last:
                rs_cw[s, q1] = acc.astype(jnp.bfloat16)
                rcw = [ncw0, start_rs_cw(s + 1, 1)]
            else:
                ag_buf[pl.ds(d * mc + qc, qc), :] = acc.astype(jnp.bfloat16)
                acw[1] = start_ag_cw(0, 1)
            rccw[1].wait_recv()
            acc = pb1 + rs_ccw[s, q1].astype(jnp.float32)
            if not last:
                rs_ccw[s, q1] = acc.astype(jnp.bfloat16)
                rccw = [nccw0, start_rs_ccw(s + 1, 1)]
            else:
                ag_buf[pl.ds(d * mc + hc + qc, qc), :] = acc.astype(
                    jnp.bfloat16)
                accw[1] = start_ag_ccw(0, 1)

        def qdot_out(c, off, q):
            return lax.dot(
                ag_buf[pl.ds(c * mc + off + q * qc, qc), :], w2_ref[...],
                preferred_element_type=jnp.float32)

        store_half(d, top=True)
        store_half(d, top=False)
        for t in range(N_DEV - 1):
            ct = lax.rem(d + 2 * N_DEV - 1 - t, N_DEV)
            cb = lax.rem(d + 1 + t, N_DEV)
            nacw, naccw = [None, None], [None, None]
            acw[0].wait_recv()
            if t < N_DEV - 2:
                nacw[0] = start_ag_cw(t + 1, 0)
            prev_store[0].wait()
            ostage[0, q0] = qdot_out(ct, 0, 0)
            accw[0].wait_recv()
            if t < N_DEV - 2:
                naccw[0] = start_ag_ccw(t + 1, 0)
            prev_store[1].wait()
            ostage[1, q0] = qdot_out(cb, hc, 0)
            acw[1].wait_recv()
            if t < N_DEV - 2:
                nacw[1] = start_ag_cw(t + 1, 1)
            ostage[0, q1] = qdot_out(ct, 0, 1)
            cpT = pltpu.make_async_copy(
                ostage.at[0], out_ref.at[pl.ds(ct * mc, hc), :],
                st_sems.at[0])
            cpT.start()
            prev_store[0] = cpT
            accw[1].wait_recv()
            if t < N_DEV - 2:
                naccw[1] = start_ag_ccw(t + 1, 1)
            ostage[1, q1] = qdot_out(cb, hc, 1)
            cpB = pltpu.make_async_copy(
                ostage.at[1], out_ref.at[pl.ds(cb * mc + hc, hc), :],
                st_sems.at[1])
            cpB.start()
            prev_store[1] = cpB
            acw, accw = nacw, naccw

        prev_store[0].wait()
        prev_store[1].wait()
        for r in sends:
            r.wait_send()

    return pl.pallas_call(
        body,
        out_shape=jax.ShapeDtypeStruct((m, f_loc), jnp.float32),
        in_specs=[
            pl.BlockSpec(memory_space=pltpu.MemorySpace.HBM),
            pl.BlockSpec(memory_space=pltpu.MemorySpace.VMEM),
            pl.BlockSpec(memory_space=pltpu.MemorySpace.VMEM),
        ],
        out_specs=pl.BlockSpec(memory_space=pltpu.MemorySpace.HBM),
        scratch_shapes=[
            pltpu.VMEM((hc, k_loc), jnp.float32),
            pltpu.VMEM((hc, k_loc), jnp.float32),
            pltpu.VMEM((2, hc, f_loc), jnp.float32),
            pltpu.VMEM((N_DEV, hc, d_model), jnp.bfloat16),
            pltpu.VMEM((N_DEV, hc, d_model), jnp.bfloat16),
            pltpu.VMEM((m, d_model), jnp.bfloat16),
            pltpu.SemaphoreType.DMA,
            pltpu.SemaphoreType.DMA,
            pltpu.SemaphoreType.DMA((2,)),
            pltpu.SemaphoreType.DMA((N_DEV - 1, 2)),
            pltpu.SemaphoreType.DMA((N_DEV - 1, 2)),
            pltpu.SemaphoreType.DMA((N_DEV - 1, 2)),
            pltpu.SemaphoreType.DMA((N_DEV - 1, 2)),
            pltpu.SemaphoreType.DMA((N_DEV - 1, 2)),
            pltpu.SemaphoreType.DMA((N_DEV - 1, 2)),
            pltpu.SemaphoreType.DMA((N_DEV - 1, 2)),
            pltpu.SemaphoreType.DMA((N_DEV - 1, 2)),
        ],
        compiler_params=pltpu.CompilerParams(
            vmem_limit_bytes=60 * 1024 * 1024,
            collective_id=0,
        ),
    )(x, w1b, w2b)


# device time: 188917 ns/iter; 2.2686x vs baseline; 1.0007x over previous
import jax
import jax.numpy as jnp
from jax import lax
from jax.experimental import pallas as pl
from jax.experimental.pallas import tpu as pltpu

N_DEV = 8


def kernel(x, W1, W2):
    m, k_loc = x.shape
    _, d_model = W1.shape
    _, f_loc = W2.shape
    mc = m // N_DEV
    hc = mc // 2
    qc = hc // 2

    w1b = W1.astype(jnp.bfloat16)
    w2b = W2.astype(jnp.bfloat16)

    def body(x_ref, w1_ref, w2_ref, out_ref,
             xst, xsb, ostage, rs_cw, rs_ccw, ag_buf,
             lt_sem, lb_sem, st_sems,
             rs_cw_ss, rs_cw_rs, rs_ccw_ss, rs_ccw_rs,
             ag_cw_ss, ag_cw_rs, ag_ccw_ss, ag_ccw_rs):
        d = lax.axis_index("i")
        right = lax.rem(d + 1, N_DEV)
        left = lax.rem(d + N_DEV - 1, N_DEV)

        sends = []

        def start_loads(c_top, c_bot):
            cp1 = pltpu.make_async_copy(
                x_ref.at[pl.ds(c_top * mc, hc), :], xst, lt_sem)
            cp2 = pltpu.make_async_copy(
                x_ref.at[pl.ds(c_bot * mc + hc, hc), :], xsb, lb_sem)
            cp1.start()
            cp2.start()
            return cp1, cp2

        def qdot(stage, q):
            return lax.dot(
                stage[pl.ds(q * qc, qc), :].astype(jnp.bfloat16),
                w1_ref[...],
                preferred_element_type=jnp.float32).astype(jnp.bfloat16)

        def start_rs_cw(s, q):
            slot = N_DEV - 1 if s == 0 else s - 1
            r = pltpu.make_async_remote_copy(
                src_ref=rs_cw.at[slot, pl.ds(q * qc, qc)],
                dst_ref=rs_cw.at[s, pl.ds(q * qc, qc)],
                send_sem=rs_cw_ss.at[s, q], recv_sem=rs_cw_rs.at[s, q],
                device_id=(right,), device_id_type=pl.DeviceIdType.MESH)
            r.start()
            sends.append(r)
            return r

        def start_rs_ccw(s, q):
            slot = N_DEV - 1 if s == 0 else s - 1
            r = pltpu.make_async_remote_copy(
                src_ref=rs_ccw.at[slot, pl.ds(q * qc, qc)],
                dst_ref=rs_ccw.at[s, pl.ds(q * qc, qc)],
                send_sem=rs_ccw_ss.at[s, q], recv_sem=rs_ccw_rs.at[s, q],
                device_id=(left,), device_id_type=pl.DeviceIdType.MESH)
            r.start()
            sends.append(r)
            return r

        def start_ag_cw(t, q):
            c = lax.rem(d + N_DEV - t, N_DEV)
            a = pltpu.make_async_remote_copy(
                src_ref=ag_buf.at[pl.ds(c * mc + q * qc, qc)],
                dst_ref=ag_buf.at[pl.ds(c * mc + q * qc, qc)],
                send_sem=ag_cw_ss.at[t, q], recv_sem=ag_cw_rs.at[t, q],
                device_id=(right,), device_id_type=pl.DeviceIdType.MESH)
            a.start()
            sends.append(a)
            return a

        def start_ag_ccw(t, q):
            c = lax.rem(d + t, N_DEV)
            a = pltpu.make_async_remote_copy(
                src_ref=ag_buf.at[pl.ds(c * mc + hc + q * qc, qc)],
                dst_ref=ag_buf.at[pl.ds(c * mc + hc + q * qc, qc)],
                send_sem=ag_ccw_ss.at[t, q], recv_sem=ag_ccw_rs.at[t, q],
                device_id=(left,), device_id_type=pl.DeviceIdType.MESH)
            a.start()
            sends.append(a)
            return a

        prev_store = [None, None]

        def store_half(c, top):
            slot = 0 if top else 1
            off = 0 if top else hc
            if prev_store[slot] is not None:
                prev_store[slot].wait()
            ostage[slot] = lax.dot(
                ag_buf[pl.ds(c * mc + off, hc), :], w2_ref[...],
                preferred_element_type=jnp.float32)
            cp = pltpu.make_async_copy(
                ostage.at[slot], out_ref.at[pl.ds(c * mc + off, hc), :],
                st_sems.at[slot])
            cp.start()
            prev_store[slot] = cp

        cp1, cp2 = start_loads(lax.rem(d + N_DEV - 1, N_DEV),
                               lax.rem(d + 1, N_DEV))

        barrier_sem = pltpu.get_barrier_semaphore()
        pl.semaphore_signal(barrier_sem, inc=1, device_id=(left,),
                            device_id_type=pl.DeviceIdType.MESH)
        pl.semaphore_signal(barrier_sem, inc=1, device_id=(right,),
                            device_id_type=pl.DeviceIdType.MESH)
        pl.semaphore_wait(barrier_sem, 2)

        cp1.wait()
        rs_cw[N_DEV - 1, pl.ds(0, qc)] = qdot(xst, 0)
        rcw = [start_rs_cw(0, 0), None]
        cp2.wait()
        rs_ccw[N_DEV - 1, pl.ds(0, qc)] = qdot(xsb, 0)
        rccw = [start_rs_ccw(0, 0), None]
        rs_cw[N_DEV - 1, pl.ds(qc, qc)] = qdot(xst, 1)
        rcw[1] = start_rs_cw(0, 1)
        rs_ccw[N_DEV - 1, pl.ds(qc, qc)] = qdot(xsb, 1)
        rccw[1] = start_rs_ccw(0, 1)

        q0, q1 = pl.ds(0, qc), pl.ds(qc, qc)
        acw = [None, None]
        accw = [None, None]
        for s in range(N_DEV - 1):
            cp1, cp2 = start_loads(lax.rem(d + 2 * N_DEV - 2 - s, N_DEV),
                                   lax.rem(d + 2 + s, N_DEV))
            cp1.wait()
            pt0 = qdot(xst, 0)
            cp2.wait()
            pb0 = qdot(xsb, 0)
            last = s == N_DEV - 2
            rcw[0].wait_recv()
            acc = pt0 + rs_cw[s, q0]
            if not last:
                rs_cw[s, q0] = acc
                ncw0 = start_rs_cw(s + 1, 0)
            else:
                ag_buf[pl.ds(d * mc, qc), :] = acc
                acw[0] = start_ag_cw(0, 0)
            rccw[0].wait_recv()
            acc = pb0 + rs_ccw[s, q0]
            if not last:
                rs_ccw[s, q0] = acc
                nccw0 = start_rs_ccw(s + 1, 0)
            else:
                ag_buf[pl.ds(d * mc + hc, qc), :] = acc
                accw[0] = start_ag_ccw(0, 0)
            pt1 = qdot(xst, 1)
            pb1 = qdot(xsb, 1)
            rcw[1].wait_recv()
            acc = pt1 + rs_cw[s, q1]
            if not last:
                rs_cw[s, q1] = acc
                rcw = [ncw0, start_rs_cw(s + 1, 1)]
            else:
                ag_buf[pl.ds(d * mc + qc, qc), :] = acc
                acw[1] = start_ag_cw(0, 1)
            rccw[1].wait_recv()
            acc = pb1 + rs_ccw[s, q1]
            if not last:
                rs_ccw[s, q1] = acc
                rccw = [nccw0, start_rs_ccw(s + 1, 1)]
            else:
                ag_buf[pl.ds(d * mc + hc + qc, qc), :] = acc
                accw[1] = start_ag_ccw(0, 1)

        def qdot_out(c, off, q):
            return lax.dot(
                ag_buf[pl.ds(c * mc + off + q * qc, qc), :], w2_ref[...],
                preferred_element_type=jnp.float32)

        store_half(d, top=True)
        store_half(d, top=False)
        for t in range(N_DEV - 1):
            ct = lax.rem(d + 2 * N_DEV - 1 - t, N_DEV)
            cb = lax.rem(d + 1 + t, N_DEV)
            nacw, naccw = [None, None], [None, None]
            acw[0].wait_recv()
            if t < N_DEV - 2:
                nacw[0] = start_ag_cw(t + 1, 0)
            prev_store[0].wait()
            ostage[0, q0] = qdot_out(ct, 0, 0)
            accw[0].wait_recv()
            if t < N_DEV - 2:
                naccw[0] = start_ag_ccw(t + 1, 0)
            prev_store[1].wait()
            ostage[1, q0] = qdot_out(cb, hc, 0)
            acw[1].wait_recv()
            if t < N_DEV - 2:
                nacw[1] = start_ag_cw(t + 1, 1)
            ostage[0, q1] = qdot_out(ct, 0, 1)
            cpT = pltpu.make_async_copy(
                ostage.at[0], out_ref.at[pl.ds(ct * mc, hc), :],
                st_sems.at[0])
            cpT.start()
            prev_store[0] = cpT
            accw[1].wait_recv()
            if t < N_DEV - 2:
                naccw[1] = start_ag_ccw(t + 1, 1)
            ostage[1, q1] = qdot_out(cb, hc, 1)
            cpB = pltpu.make_async_copy(
                ostage.at[1], out_ref.at[pl.ds(cb * mc + hc, hc), :],
                st_sems.at[1])
            cpB.start()
            prev_store[1] = cpB
            acw, accw = nacw, naccw

        prev_store[0].wait()
        prev_store[1].wait()
        for r in sends:
            r.wait_send()

    return pl.pallas_call(
        body,
        out_shape=jax.ShapeDtypeStruct((m, f_loc), jnp.float32),
        in_specs=[
            pl.BlockSpec(memory_space=pltpu.MemorySpace.HBM),
            pl.BlockSpec(memory_space=pltpu.MemorySpace.VMEM),
            pl.BlockSpec(memory_space=pltpu.MemorySpace.VMEM),
        ],
        out_specs=pl.BlockSpec(memory_space=pltpu.MemorySpace.HBM),
        scratch_shapes=[
            pltpu.VMEM((hc, k_loc), jnp.float32),
            pltpu.VMEM((hc, k_loc), jnp.float32),
            pltpu.VMEM((2, hc, f_loc), jnp.float32),
            pltpu.VMEM((N_DEV, hc, d_model), jnp.bfloat16),
            pltpu.VMEM((N_DEV, hc, d_model), jnp.bfloat16),
            pltpu.VMEM((m, d_model), jnp.bfloat16),
            pltpu.SemaphoreType.DMA,
            pltpu.SemaphoreType.DMA,
            pltpu.SemaphoreType.DMA((2,)),
            pltpu.SemaphoreType.DMA((N_DEV - 1, 2)),
            pltpu.SemaphoreType.DMA((N_DEV - 1, 2)),
            pltpu.SemaphoreType.DMA((N_DEV - 1, 2)),
            pltpu.SemaphoreType.DMA((N_DEV - 1, 2)),
            pltpu.SemaphoreType.DMA((N_DEV - 1, 2)),
            pltpu.SemaphoreType.DMA((N_DEV - 1, 2)),
            pltpu.SemaphoreType.DMA((N_DEV - 1, 2)),
            pltpu.SemaphoreType.DMA((N_DEV - 1, 2)),
        ],
        compiler_params=pltpu.CompilerParams(
            vmem_limit_bytes=60 * 1024 * 1024,
            collective_id=0,
        ),
    )(x, w1b, w2b)


# device time: 183243 ns/iter; 2.3388x vs baseline; 1.0310x over previous
import jax
import jax.numpy as jnp
from jax import lax
from jax.experimental import pallas as pl
from jax.experimental.pallas import tpu as pltpu

N_DEV = 8


def kernel(x, W1, W2):
    m, k_loc = x.shape
    _, d_model = W1.shape
    _, f_loc = W2.shape
    mc = m // N_DEV
    hc = mc // 2
    qc = hc // 2

    def body(x_ref, w1_ref, w2_ref, out_ref,
             xst, xsb, ostage, rs_cw, rs_ccw, ag_buf, w1b, w2b,
             lt_sem, lb_sem, st_sems,
             rs_cw_ss, rs_cw_rs, rs_ccw_ss, rs_ccw_rs,
             ag_cw_ss, ag_cw_rs, ag_ccw_ss, ag_ccw_rs):
        d = lax.axis_index("i")
        right = lax.rem(d + 1, N_DEV)
        left = lax.rem(d + N_DEV - 1, N_DEV)

        sends = []

        def start_loads(c_top, c_bot):
            cp1 = pltpu.make_async_copy(
                x_ref.at[pl.ds(c_top * mc, hc), :], xst, lt_sem)
            cp2 = pltpu.make_async_copy(
                x_ref.at[pl.ds(c_bot * mc + hc, hc), :], xsb, lb_sem)
            cp1.start()
            cp2.start()
            return cp1, cp2

        def qdot(stage, q):
            return lax.dot(
                stage[pl.ds(q * qc, qc), :].astype(jnp.bfloat16),
                w1b[...],
                preferred_element_type=jnp.float32).astype(jnp.bfloat16)

        def start_rs_cw(s, q):
            slot = N_DEV - 1 if s == 0 else s - 1
            r = pltpu.make_async_remote_copy(
                src_ref=rs_cw.at[slot, pl.ds(q * qc, qc)],
                dst_ref=rs_cw.at[s, pl.ds(q * qc, qc)],
                send_sem=rs_cw_ss.at[s, q], recv_sem=rs_cw_rs.at[s, q],
                device_id=(right,), device_id_type=pl.DeviceIdType.MESH)
            r.start()
            sends.append(r)
            return r

        def start_rs_ccw(s, q):
            slot = N_DEV - 1 if s == 0 else s - 1
            r = pltpu.make_async_remote_copy(
                src_ref=rs_ccw.at[slot, pl.ds(q * qc, qc)],
                dst_ref=rs_ccw.at[s, pl.ds(q * qc, qc)],
                send_sem=rs_ccw_ss.at[s, q], recv_sem=rs_ccw_rs.at[s, q],
                device_id=(left,), device_id_type=pl.DeviceIdType.MESH)
            r.start()
            sends.append(r)
            return r

        def start_ag_cw(t, q):
            c = lax.rem(d + N_DEV - t, N_DEV)
            a = pltpu.make_async_remote_copy(
                src_ref=ag_buf.at[pl.ds(c * mc + q * qc, qc)],
                dst_ref=ag_buf.at[pl.ds(c * mc + q * qc, qc)],
                send_sem=ag_cw_ss.at[t, q], recv_sem=ag_cw_rs.at[t, q],
                device_id=(right,), device_id_type=pl.DeviceIdType.MESH)
            a.start()
            sends.append(a)
            return a

        def start_ag_ccw(t, q):
            c = lax.rem(d + t, N_DEV)
            a = pltpu.make_async_remote_copy(
                src_ref=ag_buf.at[pl.ds(c * mc + hc + q * qc, qc)],
                dst_ref=ag_buf.at[pl.ds(c * mc + hc + q * qc, qc)],
                send_sem=ag_ccw_ss.at[t, q], recv_sem=ag_ccw_rs.at[t, q],
                device_id=(left,), device_id_type=pl.DeviceIdType.MESH)
            a.start()
            sends.append(a)
            return a

        prev_store = [None, None]

        def store_half(c, top):
            slot = 0 if top else 1
            off = 0 if top else hc
            if prev_store[slot] is not None:
                prev_store[slot].wait()
            ostage[slot] = lax.dot(
                ag_buf[pl.ds(c * mc + off, hc), :], w2b[...],
                preferred_element_type=jnp.float32)
            cp = pltpu.make_async_copy(
                ostage.at[slot], out_ref.at[pl.ds(c * mc + off, hc), :],
                st_sems.at[slot])
            cp.start()
            prev_store[slot] = cp

        cp1, cp2 = start_loads(lax.rem(d + N_DEV - 1, N_DEV),
                               lax.rem(d + 1, N_DEV))

        w1b[...] = w1_ref[...].astype(jnp.bfloat16)
        w2b[...] = w2_ref[...].astype(jnp.bfloat16)

        barrier_sem = pltpu.get_barrier_semaphore()
        pl.semaphore_signal(barrier_sem, inc=1, device_id=(left,),
                            device_id_type=pl.DeviceIdType.MESH)
        pl.semaphore_signal(barrier_sem, inc=1, device_id=(right,),
                            device_id_type=pl.DeviceIdType.MESH)
        pl.semaphore_wait(barrier_sem, 2)

        cp1.wait()
        rs_cw[N_DEV - 1, pl.ds(0, qc)] = qdot(xst, 0)
        rcw = [start_rs_cw(0, 0), None]
        cp2.wait()
        rs_ccw[N_DEV - 1, pl.ds(0, qc)] = qdot(xsb, 0)
        rccw = [start_rs_ccw(0, 0), None]
        rs_cw[N_DEV - 1, pl.ds(qc, qc)] = qdot(xst, 1)
        rcw[1] = start_rs_cw(0, 1)
        rs_ccw[N_DEV - 1, pl.ds(qc, qc)] = qdot(xsb, 1)
        rccw[1] = start_rs_ccw(0, 1)

        q0, q1 = pl.ds(0, qc), pl.ds(qc, qc)
        acw = [None, None]
        accw = [None, None]
        for s in range(N_DEV - 1):
            cp1, cp2 = start_loads(lax.rem(d + 2 * N_DEV - 2 - s, N_DEV),
                                   lax.rem(d + 2 + s, N_DEV))
            cp1.wait()
            pt0 = qdot(xst, 0)
            cp2.wait()
            pb0 = qdot(xsb, 0)
            last = s == N_DEV - 2
            rcw[0].wait_recv()
            acc = pt0 + rs_cw[s, q0]
            if not last:
                rs_cw[s, q0] = acc
                ncw0 = start_rs_cw(s + 1, 0)
            else:
                ag_buf[pl.ds(d * mc, qc), :] = acc
                acw[0] = start_ag_cw(0, 0)
            rccw[0].wait_recv()
            acc = pb0 + rs_ccw[s, q0]
            if not last:
                rs_ccw[s, q0] = acc
                nccw0 = start_rs_ccw(s + 1, 0)
            else:
                ag_buf[pl.ds(d * mc + hc, qc), :] = acc
                accw[0] = start_ag_ccw(0, 0)
            pt1 = qdot(xst, 1)
            pb1 = qdot(xsb, 1)
            rcw[1].wait_recv()
            acc = pt1 + rs_cw[s, q1]
            if not last:
                rs_cw[s, q1] = acc
                rcw = [ncw0, start_rs_cw(s + 1, 1)]
            else:
                ag_buf[pl.ds(d * mc + qc, qc), :] = acc
                acw[1] = start_ag_cw(0, 1)
            rccw[1].wait_recv()
            acc = pb1 + rs_ccw[s, q1]
            if not last:
                rs_ccw[s, q1] = acc
                rccw = [nccw0, start_rs_ccw(s + 1, 1)]
            else:
                ag_buf[pl.ds(d * mc + hc + qc, qc), :] = acc
                accw[1] = start_ag_ccw(0, 1)

        def qdot_out(c, off, q):
            return lax.dot(
                ag_buf[pl.ds(c * mc + off + q * qc, qc), :], w2b[...],
                preferred_element_type=jnp.float32)

        store_half(d, top=True)
        store_half(d, top=False)
        for t in range(N_DEV - 1):
            ct = lax.rem(d + 2 * N_DEV - 1 - t, N_DEV)
            cb = lax.rem(d + 1 + t, N_DEV)
            nacw, naccw = [None, None], [None, None]
            acw[0].wait_recv()
            if t < N_DEV - 2:
                nacw[0] = start_ag_cw(t + 1, 0)
            prev_store[0].wait()
            ostage[0, q0] = qdot_out(ct, 0, 0)
            accw[0].wait_recv()
            if t < N_DEV - 2:
                naccw[0] = start_ag_ccw(t + 1, 0)
            prev_store[1].wait()
            ostage[1, q0] = qdot_out(cb, hc, 0)
            acw[1].wait_recv()
            if t < N_DEV - 2:
                nacw[1] = start_ag_cw(t + 1, 1)
            ostage[0, q1] = qdot_out(ct, 0, 1)
            cpT = pltpu.make_async_copy(
                ostage.at[0], out_ref.at[pl.ds(ct * mc, hc), :],
                st_sems.at[0])
            cpT.start()
            prev_store[0] = cpT
            accw[1].wait_recv()
            if t < N_DEV - 2:
                naccw[1] = start_ag_ccw(t + 1, 1)
            ostage[1, q1] = qdot_out(cb, hc, 1)
            cpB = pltpu.make_async_copy(
                ostage.at[1], out_ref.at[pl.ds(cb * mc + hc, hc), :],
                st_sems.at[1])
            cpB.start()
            prev_store[1] = cpB
            acw, accw = nacw, naccw

        prev_store[0].wait()
        prev_store[1].wait()
        for r in sends:
            r.wait_send()

    return pl.pallas_call(
        body,
        out_shape=jax.ShapeDtypeStruct((m, f_loc), jnp.float32),
        in_specs=[
            pl.BlockSpec(memory_space=pl.ANY),
            pl.BlockSpec(memory_space=pltpu.MemorySpace.VMEM),
            pl.BlockSpec(memory_space=pltpu.MemorySpace.VMEM),
        ],
        out_specs=pl.BlockSpec(memory_space=pl.ANY),
        scratch_shapes=[
            pltpu.VMEM((hc, k_loc), jnp.float32),
            pltpu.VMEM((hc, k_loc), jnp.float32),
            pltpu.VMEM((2, hc, f_loc), jnp.float32),
            pltpu.VMEM((N_DEV, hc, d_model), jnp.bfloat16),
            pltpu.VMEM((N_DEV, hc, d_model), jnp.bfloat16),
            pltpu.VMEM((m, d_model), jnp.bfloat16),
            pltpu.VMEM((k_loc, d_model), jnp.bfloat16),
            pltpu.VMEM((d_model, f_loc), jnp.bfloat16),
            pltpu.SemaphoreType.DMA,
            pltpu.SemaphoreType.DMA,
            pltpu.SemaphoreType.DMA((2,)),
            pltpu.SemaphoreType.DMA((N_DEV - 1, 2)),
            pltpu.SemaphoreType.DMA((N_DEV - 1, 2)),
            pltpu.SemaphoreType.DMA((N_DEV - 1, 2)),
            pltpu.SemaphoreType.DMA((N_DEV - 1, 2)),
            pltpu.SemaphoreType.DMA((N_DEV - 1, 2)),
            pltpu.SemaphoreType.DMA((N_DEV - 1, 2)),
            pltpu.SemaphoreType.DMA((N_DEV - 1, 2)),
            pltpu.SemaphoreType.DMA((N_DEV - 1, 2)),
        ],
        compiler_params=pltpu.CompilerParams(
            vmem_limit_bytes=60 * 1024 * 1024,
            collective_id=0,
        ),
    )(x, W1, W2)
